# baseline (speedup 1.0000x reference)
import functools

import jax
import jax.numpy as jnp
from jax import lax
from jax.experimental import pallas as pl
from jax.experimental.pallas import tpu as pltpu
from jax.experimental.pallas import tpu_sc as plsc

_NUM = 13
_CAT = 26
_H = 32
_VOCAB = 100000


def _sc_gather(tab_flat, gidx2d, n):
    NW = 32
    per_w = n // NW
    K = 1024
    JROWS = K // 128
    nchunk = per_w // K
    mesh = plsc.VectorSubcoreMesh(core_axis_name="c", subcore_axis_name="s")

    @functools.partial(
        pl.kernel,
        mesh=mesh,
        out_type=jax.ShapeDtypeStruct((n, _H), jnp.float32),
        scratch_types=[
            pltpu.VMEM((JROWS, 128), jnp.int32),
            pltpu.VMEM((K, _H), jnp.float32),
            pltpu.SemaphoreType.DMA,
        ],
        compiler_params=pltpu.CompilerParams(use_tc_tiling_on_sc=False),
    )
    def gather_kernel(tab_hbm, gidx_hbm, out_hbm, idx_v, rows_v, sem):
        cid = lax.axis_index("c")
        sid = lax.axis_index("s")
        wid = sid * 2 + cid
        base = wid * per_w

        def chunk(i, carry):
            off = pl.multiple_of(base + i * K, K)
            row0 = pl.multiple_of(off // 128, JROWS)
            pltpu.sync_copy(gidx_hbm.at[pl.ds(row0, JROWS)], idx_v)
            cps = [
                pltpu.async_copy(
                    tab_hbm.at[idx_v.at[j]],
                    rows_v.at[pl.ds(j * 128, 128)],
                    sem,
                )
                for j in range(JROWS)
            ]
            for c in cps:
                c.wait()
            pltpu.sync_copy(rows_v, out_hbm.at[pl.ds(off, K)])
            return carry

        lax.fori_loop(0, nchunk, chunk, 0)

    return gather_kernel(tab_flat, gidx2d)


def _tc_bn(x, W_exp, b_flat, cat2d, g_n, bt_n, g_c, bt_c):
    B = x.shape[0]
    DN = W_exp.shape[1]
    DC = cat2d.shape[1]
    NB = 16
    Bb = B // NB
    inv_b = 1.0 / B

    def body(x_ref, w_ref, b_ref, cat_ref, gn_ref, bn_ref, gc_ref, bc_ref,
             out_ref, stn_ref, stc_ref):
        p = pl.program_id(0)
        i = pl.program_id(1)
        xb = x_ref[...]
        en = jnp.maximum(
            jnp.dot(xb, w_ref[...], preferred_element_type=jnp.float32)
            + b_ref[...], 0.0)
        cb = cat_ref[...]

        @pl.when(jnp.logical_and(p == 0, i == 0))
        def _init():
            stn_ref[...] = jnp.zeros_like(stn_ref)
            stc_ref[...] = jnp.zeros_like(stc_ref)

        @pl.when(p == 0)
        def _stats():
            stn_ref[0:1, :] += jnp.sum(en, axis=0, keepdims=True)
            stn_ref[1:2, :] += jnp.sum(en * en, axis=0, keepdims=True)
            stc_ref[0:1, :] += jnp.sum(cb, axis=0, keepdims=True)
            stc_ref[1:2, :] += jnp.sum(cb * cb, axis=0, keepdims=True)

        @pl.when(jnp.logical_and(p == 0, i == NB - 1))
        def _finalize():
            mean_n = stn_ref[0:1, :] * inv_b
            var_n = stn_ref[1:2, :] * inv_b - mean_n * mean_n
            sc_n = gn_ref[...] * lax.rsqrt(var_n + 1e-5)
            stn_ref[2:3, :] = sc_n
            stn_ref[3:4, :] = bn_ref[...] - mean_n * sc_n
            mean_c = stc_ref[0:1, :] * inv_b
            var_c = stc_ref[1:2, :] * inv_b - mean_c * mean_c
            sc_c = gc_ref[...] * lax.rsqrt(var_c + 1e-5)
            stc_ref[2:3, :] = sc_c
            stc_ref[3:4, :] = bc_ref[...] - mean_c * sc_c

        @pl.when(p == 1)
        def _write():
            out_ref[:, :DN] = en * stn_ref[2:3, :] + stn_ref[3:4, :]
            out_ref[:, DN:] = cb * stc_ref[2:3, :] + stc_ref[3:4, :]

    return pl.pallas_call(
        body,
        grid=(2, NB),
        in_specs=[
            pl.BlockSpec((Bb, _NUM), lambda p, i: (i, 0)),
            pl.BlockSpec((_NUM, DN), lambda p, i: (0, 0)),
            pl.BlockSpec((1, DN), lambda p, i: (0, 0)),
            pl.BlockSpec((Bb, DC), lambda p, i: (i, 0)),
            pl.BlockSpec((1, DN), lambda p, i: (0, 0)),
            pl.BlockSpec((1, DN), lambda p, i: (0, 0)),
            pl.BlockSpec((1, DC), lambda p, i: (0, 0)),
            pl.BlockSpec((1, DC), lambda p, i: (0, 0)),
        ],
        out_specs=pl.BlockSpec(
            (Bb, DN + DC), lambda p, i: (jnp.where(p == 0, 0, i), 0)),
        out_shape=jax.ShapeDtypeStruct((B, DN + DC), jnp.float32),
        scratch_shapes=[
            pltpu.VMEM((4, DN), jnp.float32),
            pltpu.VMEM((4, DC), jnp.float32),
        ],
        compiler_params=pltpu.CompilerParams(
            dimension_semantics=("arbitrary", "arbitrary")),
    )(x, W_exp, b_flat, cat2d, g_n, bt_n, g_c, bt_c)


def kernel(input_data, num_W, num_b, cat_tables, bn_gamma, bn_beta):
    B = input_data.shape[0]
    x = input_data[:, :_NUM]
    idx = input_data[:, _NUM:].astype(jnp.int32)
    gidx = idx + (jnp.arange(_CAT, dtype=jnp.int32) * _VOCAB)[None, :]
    n = B * _CAT
    gidx2d = gidx.reshape(n // 128, 128)
    tab_flat = cat_tables.reshape(_CAT * _VOCAB, _H)

    emb = _sc_gather(tab_flat, gidx2d, n)
    cat2d = emb.reshape(B, _CAT * _H)

    DN = _NUM * _H
    W_exp = (num_W[:, None, :]
             * jnp.eye(_NUM, dtype=jnp.float32)[:, :, None]).reshape(_NUM, DN)
    b_flat = num_b.reshape(1, DN)
    g_n = bn_gamma[:DN].reshape(1, DN)
    bt_n = bn_beta[:DN].reshape(1, DN)
    g_c = bn_gamma[DN:].reshape(1, _CAT * _H)
    bt_c = bn_beta[DN:].reshape(1, _CAT * _H)

    out2d = _tc_bn(x, W_exp, b_flat, cat2d, g_n, bt_n, g_c, bt_c)
    return out2d.reshape(B, _NUM + _CAT, _H)
